# baseline (speedup 1.0000x reference)
import functools

import jax
import jax.numpy as jnp
from jax import lax
from jax.experimental import pallas as pl
from jax.experimental.pallas import tpu as pltpu
from jax.experimental.pallas import tpu_sc as plsc

_NUM_FIELDS = 26
_VOCAB = 100000
_EMBED_DIM = 32
_BATCH = 4096

_NUM_CORES = 2
_NUM_SUBCORES = 16
_NUM_WORKERS = _NUM_CORES * _NUM_SUBCORES

_TOTAL_ROWS = _BATCH * _NUM_FIELDS
_ROWS_PER_WORKER = _TOTAL_ROWS // _NUM_WORKERS
_CHUNK = 128
_NUM_CHUNKS = _ROWS_PER_WORKER // _CHUNK
_LANES = 16


def _body(idx_hbm, tab_hbm, out_hbm, idx_raw_v, idx_flat_v, rows_v, sem):
    wid = lax.axis_index("s") * _NUM_CORES + lax.axis_index("c")
    base = wid * _ROWS_PER_WORKER

    pltpu.sync_copy(idx_hbm.at[pl.ds(base, _ROWS_PER_WORKER)], idx_raw_v)

    def compute(t, carry):
        q = t * _LANES
        pos = q + lax.iota(jnp.int32, _LANES)
        off = (pos % _NUM_FIELDS) * _VOCAB
        idx_flat_v[pl.ds(q, _LANES)] = idx_raw_v[pl.ds(q, _LANES)] + off
        return carry

    lax.fori_loop(0, _ROWS_PER_WORKER // _LANES, compute, 0, unroll=4)

    def fire(j, carry):
        q = j * _CHUNK
        pltpu.make_async_copy(
            tab_hbm.at[idx_flat_v.at[pl.ds(q, _CHUNK)]],
            rows_v.at[pl.ds(q, _CHUNK)],
            sem,
        ).start()
        return carry

    lax.fori_loop(0, _NUM_CHUNKS, fire, 0)

    pltpu.make_async_copy(
        out_hbm.at[pl.ds(base, _ROWS_PER_WORKER)], rows_v, sem
    ).wait()

    pltpu.sync_copy(rows_v, out_hbm.at[pl.ds(base, _ROWS_PER_WORKER)])


@jax.jit
def _embed(categorical_inputs, tables):
    idx_flat = categorical_inputs.reshape(_TOTAL_ROWS)
    tab_flat = tables.reshape(_NUM_FIELDS * _VOCAB, _EMBED_DIM)

    mesh = plsc.VectorSubcoreMesh(core_axis_name="c", subcore_axis_name="s")
    out = pl.kernel(
        _body,
        out_type=jax.ShapeDtypeStruct((_TOTAL_ROWS, _EMBED_DIM), jnp.float32),
        mesh=mesh,
        scratch_types=[
            pltpu.VMEM((_ROWS_PER_WORKER,), jnp.int32),
            pltpu.VMEM((_ROWS_PER_WORKER,), jnp.int32),
            pltpu.VMEM((_ROWS_PER_WORKER, _EMBED_DIM), jnp.float32),
            pltpu.SemaphoreType.DMA,
        ],
        compiler_params=pltpu.CompilerParams(use_tc_tiling_on_sc=False),
    )(idx_flat, tab_flat)
    return out.reshape(_BATCH, _NUM_FIELDS * _EMBED_DIM)


def kernel(categorical_inputs, tables):
    return _embed(categorical_inputs, tables)
